# Initial kernel scaffold; baseline (speedup 1.0000x reference)
import jax
import jax.numpy as jnp
from jax.experimental import pallas as pl


def kernel(x, att_l, att_r, alpha, index):
    raise NotImplementedError("write your pallas kernel here")


# speedup vs baseline: 2.1981x; 2.1981x over previous
import functools

import jax
import jax.numpy as jnp
from jax import lax
from jax.experimental import pallas as pl
from jax.experimental.pallas import tpu as pltpu
from jax.experimental.pallas import tpu_sc as plsc

E = 1600000
H = 4
D = 8
N_NODES = 100000
ATT_NORM = 4.0
EPS = 1e-12

NC = 2
NS = 16
NW = NC * NS
L = 16

RN = 50000
CN = 1024
RO = 128
RBLK = 400

NQ = 25088
NPT = NQ // NW
SUB = NPT * H
TBL = NQ * H

CE_B = 512
CE_C = 1000
EPT = E // NW


def _norm_body(x_ref, o_ref):
    x = x_ref[...]
    x2 = x * x
    r = lax.broadcasted_iota(jnp.int32, (CN, RO), 0)
    c = lax.broadcasted_iota(jnp.int32, (CN, RO), 1)
    g = (r // D == c).astype(jnp.float32)
    o_ref[...] = jnp.dot(x2, g, preferred_element_type=jnp.float32)


def _norms(x_flat):
    return pl.pallas_call(
        _norm_body,
        grid=(RN // RBLK,),
        in_specs=[pl.BlockSpec((RBLK, CN), lambda i: (i, 0))],
        out_specs=pl.BlockSpec((RBLK, RO), lambda i: (i, 0)),
        out_shape=jax.ShapeDtypeStruct((RN, RO), jnp.float32),
    )(x_flat)


def _scatter_body(params_hbm, idx_hbm, nx_hbm, f_hbm, pbuf, idx_v, nx_v, tbl, outb):
    wid = lax.axis_index("s") * NC + lax.axis_index("c")
    lane = lax.iota(jnp.int32, L)

    pltpu.sync_copy(params_hbm.at[wid], pbuf)
    pv = pbuf[...]

    def scal(j):
        return jnp.sum(jnp.where(lane == j, pv, 0))

    s_al = scal(0)
    nchunks = scal(1)
    s4 = scal(2)
    e4 = scal(3)

    def zinit(j, _):
        tbl[pl.ds(j * L, L)] = jnp.zeros((L,), jnp.float32)
        return 0

    lax.fori_loop(0, (L * SUB) // L, zinit, 0)

    lo4 = wid * SUB
    lane_base = lane * SUB
    head = lane % H
    rep = lane // H

    def chunk(k, _):
        off_e = jnp.maximum(jnp.minimum(s_al + k * CE_B, E - CE_B), 0)
        off_e = pl.multiple_of(off_e, 8)
        off4 = pl.multiple_of(off_e * H, 8)
        pltpu.sync_copy(idx_hbm.at[pl.ds(off_e, CE_B)], idx_v)
        pltpu.sync_copy(nx_hbm.at[pl.ds(off4, CE_B * H)], nx_v)
        base4 = off_e * H

        def group(i, _):
            ridx = plsc.load_gather(idx_v, [i * (L // H) + rep])
            a = (ridx * H - lo4) + head + lane_base
            p4 = base4 + i * L + lane
            msk = (p4 >= s4) & (p4 < e4)
            nx16 = nx_v[pl.ds(pl.multiple_of(i * L, L), L)]
            cur = plsc.load_gather(tbl, [a], mask=msk)
            plsc.store_scatter(tbl, [a], jnp.maximum(cur, nx16), mask=msk)
            return 0

        lax.fori_loop(0, (CE_B * H) // L, group, 0)
        return 0

    lax.fori_loop(0, nchunks, chunk, 0)

    def merge(j, _):
        m = tbl[pl.ds(j * L, L)]
        for l in range(1, L):
            m = jnp.maximum(m, tbl[pl.ds(l * SUB + j * L, L)])
        outb[pl.ds(j * L, L)] = m
        return 0

    lax.fori_loop(0, SUB // L, merge, 0)
    pltpu.sync_copy(outb, f_hbm.at[pl.ds(wid * SUB, SUB)])


def _scatter_max(params, index, nx_flat):
    mesh = plsc.VectorSubcoreMesh(
        core_axis_name="c", subcore_axis_name="s", num_cores=NC, num_subcores=NS)
    return pl.kernel(
        _scatter_body,
        out_type=jax.ShapeDtypeStruct((TBL,), jnp.float32),
        mesh=mesh,
        compiler_params=pltpu.CompilerParams(needs_layout_passes=False),
        scratch_types=[
            pltpu.VMEM((L,), jnp.int32),
            pltpu.VMEM((CE_B,), jnp.int32),
            pltpu.VMEM((CE_B * H,), jnp.float32),
            pltpu.VMEM((L * SUB,), jnp.float32),
            pltpu.VMEM((SUB,), jnp.float32),
        ],
    )(params, index, nx_flat)


def _gather_body(idx_hbm, f_hbm, g4_hbm, ftbl, idx_v, g_v):
    wid = lax.axis_index("s") * NC + lax.axis_index("c")
    lane = lax.iota(jnp.int32, L)
    rep = lane // H

    pltpu.sync_copy(f_hbm, ftbl)

    def chunk(k, _):
        base_e = wid * EPT + k * CE_C
        pltpu.sync_copy(idx_hbm.at[pl.ds(base_e, CE_C)], idx_v)

        def group(i, _):
            ridx = plsc.load_gather(idx_v, [i * (L // H) + rep])
            g_v[pl.ds(i * L, L)] = plsc.load_gather(ftbl, [ridx])
            return 0

        lax.fori_loop(0, (CE_C * H) // L, group, 0)
        pltpu.sync_copy(g_v, g4_hbm.at[pl.ds(base_e * H, CE_C * H)])
        return 0

    lax.fori_loop(0, EPT // CE_C, chunk, 0)


def _gather(index, f):
    mesh = plsc.VectorSubcoreMesh(
        core_axis_name="c", subcore_axis_name="s", num_cores=NC, num_subcores=NS)
    return pl.kernel(
        _gather_body,
        out_type=jax.ShapeDtypeStruct((E * H,), jnp.float32),
        mesh=mesh,
        compiler_params=pltpu.CompilerParams(needs_layout_passes=False),
        scratch_types=[
            pltpu.VMEM((TBL,), jnp.float32),
            pltpu.VMEM((CE_C,), jnp.int32),
            pltpu.VMEM((CE_C * H,), jnp.float32),
        ],
    )(index, f)


def _final_body(nx_ref, g_ref, al_ref, attl_ref, attr_ref, o_ref):
    attl = attl_ref[...]
    attr = attr_ref[...]
    lane = lax.broadcasted_iota(jnp.int32, (RBLK, RO), 1) % H
    na = jnp.zeros((RBLK, RO), jnp.float32)
    for h in range(H):
        s = jnp.sum(attl[0, h, :] * attl[0, h, :]) + jnp.sum(attr[0, h, :] * attr[0, h, :])
        na = jnp.where(lane == h, ATT_NORM * jnp.sqrt(s), na)
    denom = na * jnp.sqrt(g_ref[...] + nx_ref[...]) + EPS
    o_ref[...] = al_ref[...] / denom


def _final(nx, g4, alpha_flat, att_l, att_r):
    return pl.pallas_call(
        _final_body,
        grid=(RN // RBLK,),
        in_specs=[
            pl.BlockSpec((RBLK, RO), lambda i: (i, 0)),
            pl.BlockSpec((RBLK, RO), lambda i: (i, 0)),
            pl.BlockSpec((RBLK, RO), lambda i: (i, 0)),
            pl.BlockSpec((1, H, D), lambda i: (0, 0, 0)),
            pl.BlockSpec((1, H, D), lambda i: (0, 0, 0)),
        ],
        out_specs=pl.BlockSpec((RBLK, RO), lambda i: (i, 0)),
        out_shape=jax.ShapeDtypeStruct((RN, RO), jnp.float32),
    )(nx, g4, alpha_flat, att_l, att_r)


def kernel(x, att_l, att_r, alpha, index):
    x_flat = x.reshape(RN, CN)
    nx = _norms(x_flat)

    ticks = jnp.arange(NW + 1, dtype=jnp.int32) * NPT
    starts = jnp.searchsorted(index, ticks, side="left").astype(jnp.int32)
    s = starts[:NW]
    e = starts[1:]
    s_al = (s // 8) * 8
    nchunks = (e - s_al + CE_B - 1) // CE_B
    params = jnp.zeros((NW, L), jnp.int32)
    params = params.at[:, 0].set(s_al)
    params = params.at[:, 1].set(nchunks)
    params = params.at[:, 2].set(s * H)
    params = params.at[:, 3].set(e * H)

    nx_flat = nx.reshape(E * H)
    f = _scatter_max(params, index, nx_flat)
    g4 = _gather(index, f)
    out = _final(nx, g4.reshape(RN, RO), alpha.reshape(RN, RO), att_l, att_r)
    return out.reshape(E, H)


# speedup vs baseline: 12.6684x; 5.7633x over previous
import jax
import jax.numpy as jnp
from jax import lax
from jax.experimental import pallas as pl
from jax.experimental.pallas import tpu as pltpu
from jax.experimental.pallas import tpu_sc as plsc

E = 1600000
H = 4
D = 8
ATT_NORM = 4.0
EPS = 1e-12

NC = 2
NS = 16
NW = NC * NS
L = 16

BE = 2048
GRID = (E + BE - 1) // BE
RB = BE // 32

NQ = 25088
NPT = NQ // NW
SUB = NPT * H
TBL = NQ * H

CE_B = 512
CE_C = 1000
EPT = E // NW


def _norm_body(x_ref, o0, o1, o2, o3):
    x = x_ref[...]
    x2 = x * x
    outs = (o0, o1, o2, o3)
    for h in range(H):
        outs[h][...] = jnp.sum(x2[h * D:(h + 1) * D, :], axis=0)


def _norms(xt):
    return pl.pallas_call(
        _norm_body,
        grid=(GRID,),
        in_specs=[pl.BlockSpec((H * D, BE), lambda i: (0, i))],
        out_specs=[pl.BlockSpec((BE,), lambda i: (i,)) for _ in range(H)],
        out_shape=[jax.ShapeDtypeStruct((E,), jnp.float32) for _ in range(H)],
    )(xt)


def _scatter_body(params_hbm, idx_hbm, n0_hbm, n1_hbm, n2_hbm, n3_hbm, f_hbm,
                  pbuf, idx_v, n0_v, n1_v, n2_v, n3_v, tbl, outb):
    wid = lax.axis_index("s") * NC + lax.axis_index("c")
    lane = lax.iota(jnp.int32, L)

    pltpu.sync_copy(params_hbm.at[wid], pbuf)
    pv = pbuf[...]

    def scal(j):
        return jnp.sum(jnp.where(lane == j, pv, 0))

    s_al = scal(0)
    nchunks = scal(1)
    s_e = scal(2)
    e_e = scal(3)

    def zinit(j, _):
        tbl[pl.ds(j * L, L)] = jnp.zeros((L,), jnp.float32)
        return 0

    lax.fori_loop(0, (L * SUB) // L, zinit, 0)

    lo4 = wid * SUB
    lane_base = lane * SUB
    nvs = (n0_v, n1_v, n2_v, n3_v)

    def chunk(k, _):
        off_e = jnp.maximum(jnp.minimum(s_al + k * CE_B, E - CE_B), 0)
        off_e = pl.multiple_of(off_e, 8)
        pltpu.sync_copy(idx_hbm.at[pl.ds(off_e, CE_B)], idx_v)
        pltpu.sync_copy(n0_hbm.at[pl.ds(off_e, CE_B)], n0_v)
        pltpu.sync_copy(n1_hbm.at[pl.ds(off_e, CE_B)], n1_v)
        pltpu.sync_copy(n2_hbm.at[pl.ds(off_e, CE_B)], n2_v)
        pltpu.sync_copy(n3_hbm.at[pl.ds(off_e, CE_B)], n3_v)

        def group(i, _):
            sl = pl.ds(pl.multiple_of(i * L, L), L)
            eidx = idx_v[sl]
            p = off_e + i * L + lane
            msk = (p >= s_e) & (p < e_e)
            a0 = eidx * H - lo4 + lane_base
            for h in range(H):
                cur = plsc.load_gather(tbl, [a0 + h], mask=msk)
                plsc.store_scatter(tbl, [a0 + h],
                                   jnp.maximum(cur, nvs[h][sl]), mask=msk)
            return 0

        lax.fori_loop(0, CE_B // L, group, 0)
        return 0

    lax.fori_loop(0, nchunks, chunk, 0)

    def merge(j, _):
        m = tbl[pl.ds(j * L, L)]
        for l in range(1, L):
            m = jnp.maximum(m, tbl[pl.ds(l * SUB + j * L, L)])
        outb[pl.ds(j * L, L)] = m
        return 0

    lax.fori_loop(0, SUB // L, merge, 0)
    pltpu.sync_copy(outb, f_hbm.at[pl.ds(wid * SUB, SUB)])


def _scatter_max(params, index, nx_planes):
    mesh = plsc.VectorSubcoreMesh(
        core_axis_name="c", subcore_axis_name="s", num_cores=NC, num_subcores=NS)
    return pl.kernel(
        _scatter_body,
        out_type=jax.ShapeDtypeStruct((TBL,), jnp.float32),
        mesh=mesh,
        compiler_params=pltpu.CompilerParams(needs_layout_passes=False),
        scratch_types=[
            pltpu.VMEM((L,), jnp.int32),
            pltpu.VMEM((CE_B,), jnp.int32),
            pltpu.VMEM((CE_B,), jnp.float32),
            pltpu.VMEM((CE_B,), jnp.float32),
            pltpu.VMEM((CE_B,), jnp.float32),
            pltpu.VMEM((CE_B,), jnp.float32),
            pltpu.VMEM((L * SUB,), jnp.float32),
            pltpu.VMEM((SUB,), jnp.float32),
        ],
    )(params, index, *nx_planes)


NG = E // 128
NP2 = NG // 2
P_LO = NP2 // NW
P_EXTRA = NP2 - P_LO * NW
GC = 8
CC = GC * 128
NCH = ((P_LO + 1) * 2 + GC - 1) // GC


def _gather_body(idx_hbm, f_hbm, n0_hbm, n1_hbm, n2_hbm, n3_hbm, s2_hbm,
                 ftbl, idx_v, n0_v, n1_v, n2_v, n3_v, sbuf):
    wid = lax.axis_index("s") * NC + lax.axis_index("c")

    pltpu.sync_copy(f_hbm, ftbl)

    base_g = (wid * P_LO + jnp.minimum(wid, P_EXTRA)) * 2
    ng = (P_LO + jnp.where(wid < P_EXTRA, 1, 0)) * 2
    nvs = (n0_v, n1_v, n2_v, n3_v)

    def chunk(k, _):
        off_g = jnp.minimum(base_g + k * GC, base_g + ng - GC)
        off_e = pl.multiple_of(off_g * 128, 8)
        pltpu.sync_copy(idx_hbm.at[pl.ds(off_e, CC)], idx_v)
        pltpu.sync_copy(n0_hbm.at[pl.ds(off_e, CC)], n0_v)
        pltpu.sync_copy(n1_hbm.at[pl.ds(off_e, CC)], n1_v)
        pltpu.sync_copy(n2_hbm.at[pl.ds(off_e, CC)], n2_v)
        pltpu.sync_copy(n3_hbm.at[pl.ds(off_e, CC)], n3_v)

        for j in range(GC):
            def ii_body(ii, _):
                sl = pl.ds(pl.multiple_of(j * 128 + ii * L, L), L)
                gv = plsc.load_gather(ftbl, [idx_v[sl]])
                c16 = pl.ds(pl.multiple_of(ii * L, L), L)
                for h in range(H):
                    sbuf[H * j + h, c16] = gv + nvs[h][sl]
                return 0

            lax.fori_loop(0, 128 // L, ii_body, 0)

        row0 = pl.multiple_of(off_g * H, 8)
        pltpu.sync_copy(sbuf, s2_hbm.at[pl.ds(row0, H * GC), :])
        return 0

    lax.fori_loop(0, NCH, chunk, 0)


def _gather_fuse(index, f, nx_planes):
    mesh = plsc.VectorSubcoreMesh(
        core_axis_name="c", subcore_axis_name="s", num_cores=NC, num_subcores=NS)
    return pl.kernel(
        _gather_body,
        out_type=jax.ShapeDtypeStruct((E // 128 * H, 128), jnp.float32),
        mesh=mesh,
        compiler_params=pltpu.CompilerParams(needs_layout_passes=False),
        scratch_types=[
            pltpu.VMEM((TBL,), jnp.float32),
            pltpu.VMEM((CC,), jnp.int32),
            pltpu.VMEM((CC,), jnp.float32),
            pltpu.VMEM((CC,), jnp.float32),
            pltpu.VMEM((CC,), jnp.float32),
            pltpu.VMEM((CC,), jnp.float32),
            pltpu.VMEM((H * GC, 128), jnp.float32),
        ],
    )(index, f, *nx_planes)


RD = 80


def _final_body(a_ref, s_ref, attl_ref, attr_ref, o_ref):
    attl = attl_ref[...]
    attr = attr_ref[...]
    hrow = lax.broadcasted_iota(jnp.int32, (RD, 128), 0) % H
    na = jnp.zeros((RD, 128), jnp.float32)
    for h in range(H):
        s = jnp.sum(attl[0, h, :] * attl[0, h, :]) + jnp.sum(attr[0, h, :] * attr[0, h, :])
        na = jnp.where(hrow == h, ATT_NORM * jnp.sqrt(s), na)
    o_ref[...] = a_ref[...] / (na * jnp.sqrt(s_ref[...]) + EPS)


def _final(a2, s2, att_l, att_r):
    return pl.pallas_call(
        _final_body,
        grid=(E // 128 * H // RD,),
        in_specs=[
            pl.BlockSpec((RD, 128), lambda i: (i, 0)),
            pl.BlockSpec((RD, 128), lambda i: (i, 0)),
            pl.BlockSpec((1, H, D), lambda i: (0, 0, 0)),
            pl.BlockSpec((1, H, D), lambda i: (0, 0, 0)),
        ],
        out_specs=pl.BlockSpec((RD, 128), lambda i: (i, 0)),
        out_shape=jax.ShapeDtypeStruct((E // 128 * H, 128), jnp.float32),
    )(a2, s2, att_l, att_r)


def kernel(x, att_l, att_r, alpha, index):
    xt = jnp.transpose(x, (1, 2, 0)).reshape(H * D, E)
    a2 = jnp.transpose(alpha.reshape(E // 128, 128, H), (0, 2, 1)).reshape(E // 128 * H, 128)

    nx_planes = _norms(xt)

    ticks = jnp.arange(NW + 1, dtype=jnp.int32) * NPT
    starts = jnp.searchsorted(index, ticks, side="left").astype(jnp.int32)
    s = starts[:NW]
    e = starts[1:]
    s_al = (s // 8) * 8
    nchunks = (e - s_al + CE_B - 1) // CE_B
    params = jnp.zeros((NW, L), jnp.int32)
    params = params.at[:, 0].set(s_al)
    params = params.at[:, 1].set(nchunks)
    params = params.at[:, 2].set(s)
    params = params.at[:, 3].set(e)

    f = _scatter_max(params, index, nx_planes)
    s2 = _gather_fuse(index, f, nx_planes)
    out2 = _final(a2, s2, att_l, att_r)
    return jnp.transpose(out2.reshape(E // 128, H, 128), (0, 2, 1)).reshape(E, H)


# speedup vs baseline: 22.4538x; 1.7724x over previous
import jax
import jax.numpy as jnp
from jax import lax
from jax.experimental import pallas as pl
from jax.experimental.pallas import tpu as pltpu
from jax.experimental.pallas import tpu_sc as plsc

E = 1600000
H = 4
D = 8
ATT_NORM = 4.0
EPS = 1e-12

NC = 2
NS = 16
NW = NC * NS
L = 16

BE = 8192
GRID = (E + BE - 1) // BE
RB = BE // 32

NQ = 25088
NPT = NQ // NW
SUB = NPT * H
TBL = NQ * H

CE_B = 512
CE_C = 1000
EPT = E // NW


def _norm_body(x_ref, o0, o1, o2, o3):
    x = x_ref[...]
    x2 = x * x
    outs = (o0, o1, o2, o3)
    for h in range(H):
        outs[h][...] = jnp.sum(x2[h * D:(h + 1) * D, :], axis=0)


def _norms(xt):
    return pl.pallas_call(
        _norm_body,
        grid=(GRID,),
        in_specs=[pl.BlockSpec((H * D, BE), lambda i: (0, i))],
        out_specs=[pl.BlockSpec((BE,), lambda i: (i,)) for _ in range(H)],
        out_shape=[jax.ShapeDtypeStruct((E,), jnp.float32) for _ in range(H)],
    )(xt)


def _scatter_body(params_hbm, idx_hbm, n0_hbm, n1_hbm, n2_hbm, n3_hbm, f_hbm,
                  pbuf, idx_v, n0_v, n1_v, n2_v, n3_v, tbl, outb):
    wid = lax.axis_index("s") * NC + lax.axis_index("c")
    lane = lax.iota(jnp.int32, L)

    pltpu.sync_copy(params_hbm.at[wid], pbuf)
    pv = pbuf[...]

    def scal(j):
        return jnp.sum(jnp.where(lane == j, pv, 0))

    s_al = scal(0)
    nchunks = scal(1)
    s_e = scal(2)
    e_e = scal(3)

    def zinit(j, _):
        tbl[pl.ds(j * L, L)] = jnp.zeros((L,), jnp.float32)
        return 0

    lax.fori_loop(0, (L * SUB) // L, zinit, 0)

    lo4 = wid * SUB
    lane_base = lane * SUB
    nvs = (n0_v, n1_v, n2_v, n3_v)

    def chunk(k, _):
        off_e = jnp.maximum(jnp.minimum(s_al + k * CE_B, E - CE_B), 0)
        off_e = pl.multiple_of(off_e, 8)
        pltpu.sync_copy(idx_hbm.at[pl.ds(off_e, CE_B)], idx_v)
        pltpu.sync_copy(n0_hbm.at[pl.ds(off_e, CE_B)], n0_v)
        pltpu.sync_copy(n1_hbm.at[pl.ds(off_e, CE_B)], n1_v)
        pltpu.sync_copy(n2_hbm.at[pl.ds(off_e, CE_B)], n2_v)
        pltpu.sync_copy(n3_hbm.at[pl.ds(off_e, CE_B)], n3_v)

        def group(i, _):
            sl = pl.ds(pl.multiple_of(i * L, L), L)
            eidx = idx_v[sl]
            p = off_e + i * L + lane
            msk = (p >= s_e) & (p < e_e)
            a0 = eidx * H - lo4 + lane_base
            for h in range(H):
                cur = plsc.load_gather(tbl, [a0 + h], mask=msk)
                plsc.store_scatter(tbl, [a0 + h],
                                   jnp.maximum(cur, nvs[h][sl]), mask=msk)
            return 0

        lax.fori_loop(0, CE_B // L, group, 0)
        return 0

    lax.fori_loop(0, nchunks, chunk, 0)

    def merge(j, _):
        m = tbl[pl.ds(j * L, L)]
        for l in range(1, L):
            m = jnp.maximum(m, tbl[pl.ds(l * SUB + j * L, L)])
        outb[pl.ds(j * L, L)] = m
        return 0

    lax.fori_loop(0, SUB // L, merge, 0)
    pltpu.sync_copy(outb, f_hbm.at[pl.ds(wid * SUB, SUB)])


def _scatter_max(params, index, nx_planes):
    mesh = plsc.VectorSubcoreMesh(
        core_axis_name="c", subcore_axis_name="s", num_cores=NC, num_subcores=NS)
    return pl.kernel(
        _scatter_body,
        out_type=jax.ShapeDtypeStruct((TBL,), jnp.float32),
        mesh=mesh,
        compiler_params=pltpu.CompilerParams(needs_layout_passes=False),
        scratch_types=[
            pltpu.VMEM((L,), jnp.int32),
            pltpu.VMEM((CE_B,), jnp.int32),
            pltpu.VMEM((CE_B,), jnp.float32),
            pltpu.VMEM((CE_B,), jnp.float32),
            pltpu.VMEM((CE_B,), jnp.float32),
            pltpu.VMEM((CE_B,), jnp.float32),
            pltpu.VMEM((L * SUB,), jnp.float32),
            pltpu.VMEM((SUB,), jnp.float32),
        ],
    )(params, index, *nx_planes)


NG = E // 128
NP2 = NG // 2
P_LO = NP2 // NW
P_EXTRA = NP2 - P_LO * NW
GC = 8
CC = GC * 128
NCH = ((P_LO + 1) * 2 + GC - 1) // GC


def _gather_body(idx_hbm, f_hbm, n0_hbm, n1_hbm, n2_hbm, n3_hbm, s2_hbm,
                 ftbl, idx_v, n0_v, n1_v, n2_v, n3_v, sbuf):
    wid = lax.axis_index("s") * NC + lax.axis_index("c")

    pltpu.sync_copy(f_hbm, ftbl)

    base_g = (wid * P_LO + jnp.minimum(wid, P_EXTRA)) * 2
    ng = (P_LO + jnp.where(wid < P_EXTRA, 1, 0)) * 2
    nvs = (n0_v, n1_v, n2_v, n3_v)

    def chunk(k, _):
        off_g = jnp.minimum(base_g + k * GC, base_g + ng - GC)
        off_e = pl.multiple_of(off_g * 128, 8)
        pltpu.sync_copy(idx_hbm.at[pl.ds(off_e, CC)], idx_v)
        pltpu.sync_copy(n0_hbm.at[pl.ds(off_e, CC)], n0_v)
        pltpu.sync_copy(n1_hbm.at[pl.ds(off_e, CC)], n1_v)
        pltpu.sync_copy(n2_hbm.at[pl.ds(off_e, CC)], n2_v)
        pltpu.sync_copy(n3_hbm.at[pl.ds(off_e, CC)], n3_v)

        for j in range(GC):
            def ii_body(ii, _):
                sl = pl.ds(pl.multiple_of(j * 128 + ii * L, L), L)
                gv = plsc.load_gather(ftbl, [idx_v[sl]])
                c16 = pl.ds(pl.multiple_of(ii * L, L), L)
                for h in range(H):
                    sbuf[H * j + h, c16] = gv + nvs[h][sl]
                return 0

            lax.fori_loop(0, 128 // L, ii_body, 0)

        row0 = pl.multiple_of(off_g * H, 8)
        pltpu.sync_copy(sbuf, s2_hbm.at[pl.ds(row0, H * GC), :])
        return 0

    lax.fori_loop(0, NCH, chunk, 0)


def _gather_fuse(index, f, nx_planes):
    mesh = plsc.VectorSubcoreMesh(
        core_axis_name="c", subcore_axis_name="s", num_cores=NC, num_subcores=NS)
    return pl.kernel(
        _gather_body,
        out_type=jax.ShapeDtypeStruct((E // 128 * H, 128), jnp.float32),
        mesh=mesh,
        compiler_params=pltpu.CompilerParams(needs_layout_passes=False),
        scratch_types=[
            pltpu.VMEM((TBL,), jnp.float32),
            pltpu.VMEM((CC,), jnp.int32),
            pltpu.VMEM((CC,), jnp.float32),
            pltpu.VMEM((CC,), jnp.float32),
            pltpu.VMEM((CC,), jnp.float32),
            pltpu.VMEM((CC,), jnp.float32),
            pltpu.VMEM((H * GC, 128), jnp.float32),
        ],
    )(index, f, *nx_planes)


RD = 400


def _final_body(a_ref, s_ref, attl_ref, attr_ref, o_ref):
    attl = attl_ref[...]
    attr = attr_ref[...]
    hrow = lax.broadcasted_iota(jnp.int32, (RD, 128), 0) % H
    na = jnp.zeros((RD, 128), jnp.float32)
    for h in range(H):
        s = jnp.sum(attl[0, h, :] * attl[0, h, :]) + jnp.sum(attr[0, h, :] * attr[0, h, :])
        na = jnp.where(hrow == h, ATT_NORM * jnp.sqrt(s), na)
    o_ref[...] = a_ref[...] / (na * jnp.sqrt(s_ref[...]) + EPS)


def _final(a2, s2, att_l, att_r):
    return pl.pallas_call(
        _final_body,
        grid=(E // 128 * H // RD,),
        in_specs=[
            pl.BlockSpec((RD, 128), lambda i: (i, 0)),
            pl.BlockSpec((RD, 128), lambda i: (i, 0)),
            pl.BlockSpec((1, H, D), lambda i: (0, 0, 0)),
            pl.BlockSpec((1, H, D), lambda i: (0, 0, 0)),
        ],
        out_specs=pl.BlockSpec((RD, 128), lambda i: (i, 0)),
        out_shape=jax.ShapeDtypeStruct((E // 128 * H, 128), jnp.float32),
    )(a2, s2, att_l, att_r)


def kernel(x, att_l, att_r, alpha, index):
    xt = jnp.transpose(x, (1, 2, 0)).reshape(H * D, E)
    a2 = jnp.transpose(alpha.reshape(E // 128, 128, H), (0, 2, 1)).reshape(E // 128 * H, 128)

    nx_planes = _norms(xt)

    ticks = jnp.arange(NW + 1, dtype=jnp.int32) * NPT
    starts = jnp.searchsorted(index, ticks, side="left").astype(jnp.int32)
    s = starts[:NW]
    e = starts[1:]
    s_al = (s // 8) * 8
    nchunks = (e - s_al + CE_B - 1) // CE_B
    params = jnp.zeros((NW, L), jnp.int32)
    params = params.at[:, 0].set(s_al)
    params = params.at[:, 1].set(nchunks)
    params = params.at[:, 2].set(s)
    params = params.at[:, 3].set(e)

    f = _scatter_max(params, index, nx_planes)
    s2 = _gather_fuse(index, f, nx_planes)
    out2 = _final(a2, s2, att_l, att_r)
    return jnp.transpose(out2.reshape(E // 128, H, 128), (0, 2, 1)).reshape(E, H)
